# Initial kernel scaffold; baseline (speedup 1.0000x reference)
import jax
import jax.numpy as jnp
from jax.experimental import pallas as pl


def kernel(x, row_embed, col_embed):
    raise NotImplementedError("write your pallas kernel here")


# speedup vs baseline: 1.0935x; 1.0935x over previous
import jax
import jax.numpy as jnp
from jax.experimental import pallas as pl

GRID = 32
D_MODEL = 1024
BATCH = 64
HALF = D_MODEL // 2


def _pe_body(row_ref, col_ref, out_ref):
    col = col_ref[...]
    row = row_ref[...]
    first = jnp.broadcast_to(col[None, :, :], (GRID, GRID, HALF))
    second = jnp.broadcast_to(row[:, None, :], (GRID, GRID, HALF))
    out_ref[0] = jnp.concatenate([first, second], axis=-1)


def kernel(x, row_embed, col_embed):
    b = x.shape[0]
    out4 = pl.pallas_call(
        _pe_body,
        grid=(b,),
        in_specs=[
            pl.BlockSpec((GRID, HALF), lambda i: (0, 0)),
            pl.BlockSpec((GRID, HALF), lambda i: (0, 0)),
        ],
        out_specs=pl.BlockSpec((1, GRID, GRID, D_MODEL), lambda i: (i, 0, 0, 0)),
        out_shape=jax.ShapeDtypeStruct((b, GRID, GRID, D_MODEL), jnp.float32),
    )(row_embed, col_embed)
    return out4.reshape(b, GRID * GRID, D_MODEL)
